# baseline (speedup 1.0000x reference)
import functools

import jax
import jax.numpy as jnp
from jax import lax
from jax.experimental import pallas as pl
from jax.experimental.pallas import tpu as pltpu
from jax.experimental.pallas import tpu_sc as plsc

B = 4096
S = 50
D = 64
L = 16
V = 1000000
NC = 2
NS = 16
NW = NC * NS
BW = B // NW
CB = 8
NCH = BW // CB
CHI = CB * S
PIECES = ((0, 128), (128, 128), (256, 128), (384, 16))

VB = 128
NFULL = V // VB
VREM = V - NFULL * VB
BASE_BLOCKS = NFULL // NW
EXTRA_W = NFULL - BASE_BLOCKS * NW

_MESH = plsc.VectorSubcoreMesh(
    core_axis_name="c", subcore_axis_name="s", num_cores=NC, num_subcores=NS
)


def _transpose_block(slab_v, rows_v, nv, fidx):
    for v in range(nv):
        col = jnp.full((L,), v, jnp.int32)
        for fb in range(D // L):
            vals = plsc.load_gather(slab_v, [fidx + fb * L, col])
            rows_v[v // 2, pl.ds((v % 2) * D + fb * L, L)] = vals


@functools.partial(
    pl.kernel,
    out_type=jax.ShapeDtypeStruct((V * D // 128, 128), jnp.float32),
    mesh=_MESH,
    scratch_types=[
        pltpu.VMEM((D, VB), jnp.float32),
        pltpu.VMEM((VB // 2, 128), jnp.float32),
    ],
    compiler_params=pltpu.CompilerParams(needs_layout_passes=False),
)
def _relayout(tt_hbm, tail_hbm, out_hbm, slab_v, rows_v):
    wid = lax.axis_index("s") * NC + lax.axis_index("c")
    nblk = BASE_BLOCKS + jnp.where(wid < EXTRA_W, 1, 0)
    fidx = lax.iota(jnp.int32, L)

    def blk(k, carry):
        vb = wid + k * NW
        off = pl.multiple_of(vb * VB, VB)
        pltpu.sync_copy(tt_hbm.at[:, pl.ds(off, VB)], slab_v)
        _transpose_block(slab_v, rows_v, VB, fidx)
        pltpu.sync_copy(rows_v, out_hbm.at[pl.ds(vb * (VB // 2), VB // 2), :])
        return carry

    lax.fori_loop(0, nblk, blk, 0)

    @pl.when(wid == EXTRA_W)
    def _tail():
        pltpu.sync_copy(tail_hbm, rows_v.at[pl.ds(0, VREM // 2), :])
        pltpu.sync_copy(
            rows_v.at[pl.ds(0, VREM // 2), :],
            out_hbm.at[pl.ds(NFULL * (VB // 2), VREM // 2), :],
        )


def _body(idx_hbm, table_hbm, out_hbm, idx_v, buf_v, outc_v, sem):
    wid = lax.axis_index("s") * NC + lax.axis_index("c")
    base = wid * (BW * S)
    pltpu.sync_copy(idx_hbm.at[pl.ds(base, BW * S)], idx_v)

    def chunk(c, carry):
        coff = c * CHI
        descs = []
        for off, n in PIECES:
            descs.append(
                pltpu.async_copy(
                    table_hbm.at[idx_v.at[pl.ds(coff + off, n)]],
                    buf_v.at[pl.ds(off, n)],
                    sem,
                )
            )
        for d in descs:
            d.wait()
        for r in range(CB):
            for dd in range(D // L):
                acc = buf_v[r * S, pl.ds(dd * L, L)]
                for j in range(1, S):
                    acc = acc + buf_v[r * S + j, pl.ds(dd * L, L)]
                outc_v[r, pl.ds(dd * L, L)] = acc * (1.0 / S)
        pltpu.sync_copy(outc_v, out_hbm.at[pl.ds(wid * BW + c * CB, CB)])
        return carry

    lax.fori_loop(0, NCH, chunk, 0)


@functools.partial(
    pl.kernel,
    out_type=jax.ShapeDtypeStruct((B, D), jnp.float32),
    mesh=_MESH,
    scratch_types=[
        pltpu.VMEM((BW * S,), jnp.int32),
        pltpu.VMEM((CHI, D), jnp.float32),
        pltpu.VMEM((CB, D), jnp.float32),
        pltpu.SemaphoreType.DMA,
    ],
    compiler_params=pltpu.CompilerParams(use_tc_tiling_on_sc=False),
)
def _embed_mean(idx_hbm, table_hbm, out_hbm, idx_v, buf_v, outc_v, sem):
    _body(idx_hbm, table_hbm, out_hbm, idx_v, buf_v, outc_v, sem)


def kernel(inputs, table):
    idx_flat = inputs.astype(jnp.int32).reshape(-1)
    tail2 = table[NFULL * VB :, :].reshape(VREM // 2, 128)
    t2 = _relayout(table.T, tail2)
    return _embed_mean(idx_flat, t2.reshape(V, D))


# speedup vs baseline: 2.8080x; 2.8080x over previous
import functools

import jax
import jax.numpy as jnp
from jax import lax
from jax.experimental import pallas as pl
from jax.experimental.pallas import tpu as pltpu
from jax.experimental.pallas import tpu_sc as plsc

B = 4096
S = 50
D = 64
L = 16
NC = 2
NS = 16
NW = NC * NS
BW = B // NW
CB = 8
NCH = BW // CB
CHI = CB * S
PIECES = ((0, 128), (128, 128), (256, 128), (384, 16))

_MESH = plsc.VectorSubcoreMesh(
    core_axis_name="c", subcore_axis_name="s", num_cores=NC, num_subcores=NS
)


def _body(idx_hbm, table_hbm, out_hbm, idx_v, buf_v, outc_v, sem):
    wid = lax.axis_index("s") * NC + lax.axis_index("c")
    base = wid * (BW * S)
    pltpu.sync_copy(idx_hbm.at[pl.ds(base, BW * S)], idx_v)

    def chunk(c, carry):
        coff = c * CHI
        descs = []
        for off, n in PIECES:
            descs.append(
                pltpu.async_copy(
                    table_hbm.at[idx_v.at[pl.ds(coff + off, n)]],
                    buf_v.at[pl.ds(off, n)],
                    sem,
                )
            )
        for d in descs:
            d.wait()
        for r in range(CB):
            for dd in range(D // L):
                acc = buf_v[r * S, pl.ds(dd * L, L)]
                for j in range(1, S):
                    acc = acc + buf_v[r * S + j, pl.ds(dd * L, L)]
                outc_v[r, pl.ds(dd * L, L)] = acc * (1.0 / S)
        pltpu.sync_copy(outc_v, out_hbm.at[pl.ds(wid * BW + c * CB, CB)])
        return carry

    lax.fori_loop(0, NCH, chunk, 0)


@functools.partial(
    pl.kernel,
    out_type=jax.ShapeDtypeStruct((B, D), jnp.float32),
    mesh=_MESH,
    scratch_types=[
        pltpu.VMEM((BW * S,), jnp.int32),
        pltpu.VMEM((CHI, D), jnp.float32),
        pltpu.VMEM((CB, D), jnp.float32),
        pltpu.SemaphoreType.DMA,
    ],
    compiler_params=pltpu.CompilerParams(use_tc_tiling_on_sc=False),
)
def _embed_mean(idx_hbm, table_hbm, out_hbm, idx_v, buf_v, outc_v, sem):
    _body(idx_hbm, table_hbm, out_hbm, idx_v, buf_v, outc_v, sem)


def kernel(inputs, table):
    idx_flat = inputs.astype(jnp.int32).reshape(-1)
    return _embed_mean(idx_flat, table)


# speedup vs baseline: 3.0154x; 1.0738x over previous
import functools

import jax
import jax.numpy as jnp
from jax import lax
from jax.experimental import pallas as pl
from jax.experimental.pallas import tpu as pltpu
from jax.experimental.pallas import tpu_sc as plsc

B = 4096
S = 50
D = 64
L = 16
NC = 2
NS = 16
NW = NC * NS
BW = B // NW
CB = 8
NCH = BW // CB
CHI = CB * S
PIECES = ((0, 128), (128, 128), (256, 128), (384, 16))

_MESH = plsc.VectorSubcoreMesh(
    core_axis_name="c", subcore_axis_name="s", num_cores=NC, num_subcores=NS
)


def _body(idx_hbm, table_hbm, out_hbm, idx_v, buf_v, outc_v, sem):
    wid = lax.axis_index("s") * NC + lax.axis_index("c")
    base = wid * (BW * S)
    pltpu.sync_copy(idx_hbm.at[pl.ds(base, BW * S)], idx_v)

    def chunk(c, carry):
        coff = c * CHI
        descs = []
        for off, n in PIECES:
            descs.append(
                pltpu.async_copy(
                    table_hbm.at[idx_v.at[pl.ds(coff + off, n)]],
                    buf_v.at[pl.ds(off, n)],
                    sem,
                )
            )
        for d in descs:
            d.wait()
        for r in range(CB):
            for dd in range(D // L):
                acc = buf_v[r * S, pl.ds(dd * L, L)]
                for j in range(1, S):
                    acc = acc + buf_v[r * S + j, pl.ds(dd * L, L)]
                outc_v[r, pl.ds(dd * L, L)] = acc * (1.0 / S)
        pltpu.sync_copy(outc_v, out_hbm.at[pl.ds(wid * BW + c * CB, CB)])
        return carry

    lax.fori_loop(0, NCH, chunk, 0)


@functools.partial(
    pl.kernel,
    out_type=jax.ShapeDtypeStruct((B, D), jnp.float32),
    mesh=_MESH,
    scratch_types=[
        pltpu.VMEM((BW * S,), jnp.int32),
        pltpu.VMEM((CHI, 2 * D), jnp.float32),
        pltpu.VMEM((CB, D), jnp.float32),
        pltpu.SemaphoreType.DMA,
    ],
    compiler_params=pltpu.CompilerParams(use_tc_tiling_on_sc=False),
)
def _embed_mean(idx_hbm, table_hbm, out_hbm, idx_v, buf_v, outc_v, sem):
    _body(idx_hbm, table_hbm, out_hbm, idx_v, buf_v, outc_v, sem)


def kernel(inputs, table):
    idx_flat = inputs.astype(jnp.int32).reshape(-1)
    tp = jnp.pad(table, ((0, 0), (0, D)))
    return _embed_mean(idx_flat, tp)
